# baseline (speedup 1.0000x reference)
import jax
import jax.numpy as jnp
from jax import lax
from jax.experimental import pallas as pl
from jax.experimental.pallas import tpu as pltpu
from jax.experimental.pallas import tpu_sc as plsc

_NC = 2
_NS = 16
_NW = _NC * _NS
_BATCH = 16384
_D = 32
_BPW = _BATCH // _NW
_CHUNK = 128
_NCHUNK = _BPW // _CHUNK


def _sc_body(uidx_hbm, pidx_hbm, nidx_hbm, ue_hbm, ie_hbm, out_hbm,
             iu, ip, im, ru, rp, rn, ov, sem):
    wid = lax.axis_index("s") * _NC + lax.axis_index("c")
    rbase = wid * _NCHUNK

    pltpu.sync_copy(uidx_hbm.at[pl.ds(rbase, _NCHUNK)], iu)
    pltpu.sync_copy(pidx_hbm.at[pl.ds(rbase, _NCHUNK)], ip)
    pltpu.sync_copy(nidx_hbm.at[pl.ds(rbase, _NCHUNK)], im)

    copies = []
    for t in range(_NCHUNK):
        dst = pl.ds(t * _CHUNK, _CHUNK)
        copies.append(pltpu.async_copy(ue_hbm.at[iu.at[t]], ru.at[dst], sem))
        copies.append(pltpu.async_copy(ie_hbm.at[ip.at[t]], rp.at[dst], sem))
        copies.append(pltpu.async_copy(ie_hbm.at[im.at[t]], rn.at[dst], sem))
    for c in copies:
        c.wait()

    def group(g, carry):
        rows = g * 16 + lax.iota(jnp.int32, 16)
        acc_p = jnp.zeros((16,), jnp.float32)
        acc_n = jnp.zeros((16,), jnp.float32)
        for j in range(_D):
            col = jnp.full((16,), j, jnp.int32)
            u = plsc.load_gather(ru, [rows, col])
            p = plsc.load_gather(rp, [rows, col])
            n = plsc.load_gather(rn, [rows, col])
            acc_p = acc_p + u * p
            acc_n = acc_n + u * n
        ov[pl.ds(g * 16, 16)] = acc_n - acc_p
        return carry

    lax.fori_loop(0, _BPW // 16, group, 0)
    pltpu.sync_copy(ov, out_hbm.at[pl.ds(wid * _BPW, _BPW)])


@jax.jit
def _sc_diffs(uidx, pidx, nidx, ue, ie):
    mesh = plsc.VectorSubcoreMesh(core_axis_name="c", subcore_axis_name="s")
    return pl.kernel(
        _sc_body,
        out_type=jax.ShapeDtypeStruct((_BATCH,), jnp.float32),
        mesh=mesh,
        compiler_params=pltpu.CompilerParams(
            needs_layout_passes=False, use_tc_tiling_on_sc=False),
        scratch_types=[
            pltpu.VMEM((_NCHUNK, _CHUNK), jnp.int32),
            pltpu.VMEM((_NCHUNK, _CHUNK), jnp.int32),
            pltpu.VMEM((_NCHUNK, _CHUNK), jnp.int32),
            pltpu.VMEM((_BPW, _D), jnp.float32),
            pltpu.VMEM((_BPW, _D), jnp.float32),
            pltpu.VMEM((_BPW, _D), jnp.float32),
            pltpu.VMEM((_BPW,), jnp.float32),
            pltpu.SemaphoreType.DMA,
        ],
    )(uidx, pidx, nidx, ue, ie)


def _tc_body(x_ref, o_ref):
    x = x_ref[...]
    sp = jnp.maximum(x, 0.0) + jnp.log(1.0 + jnp.exp(-jnp.abs(x)))
    o_ref[...] = jnp.sum(sp, keepdims=True) * (1.0 / _BATCH)


@jax.jit
def _softplus_mean(diffs):
    out = pl.pallas_call(
        _tc_body,
        out_shape=jax.ShapeDtypeStruct((1, 1), jnp.float32),
    )(diffs.reshape(128, 128))
    return out[0, 0]


def kernel(users, positive_items, negative_items, user_embedding, item_embedding):
    uidx = users.reshape(128, 128)
    pidx = positive_items.reshape(128, 128)
    nidx = negative_items.reshape(128, 128)
    diffs = _sc_diffs(uidx, pidx, nidx, user_embedding, item_embedding)
    return _softplus_mean(diffs)
